# Initial kernel scaffold; baseline (speedup 1.0000x reference)
import functools

import jax
import jax.numpy as jnp
from jax import lax
from jax.experimental import pallas as pl
from jax.experimental.pallas import tpu as pltpu
from jax.experimental.pallas import tpu_sc as plsc

_N = 10000
_E = 320000
_D = 128
_G = 64
_H = 64
_NP = 10240
_BN = 1024
_NBLK = _NP // _BN
_NSUB = 16
_SLAB = _NP // _NSUB
_EPT = _E // _NSUB
_CH = 1000
_NCH = _EPT // _CH

_mesh = plsc.VectorSubcoreMesh(core_axis_name="c", subcore_axis_name="s")


def _deg_body(dst_hbm, ones_hbm, zeros_hbm, deg_hbm, tbl, dstv, onesv):
    c = lax.axis_index("c")
    s = lax.axis_index("s")

    @pl.when(c == 0)
    def _():
        r0 = s * _SLAB
        pltpu.sync_copy(zeros_hbm.at[pl.ds(r0, _SLAB)], tbl.at[pl.ds(r0, _SLAB)])
        pltpu.sync_copy(ones_hbm, onesv)
        plsc.subcore_barrier()
        e0 = s * _EPT
        for i in range(_NCH):
            pltpu.sync_copy(dst_hbm.at[pl.ds(e0 + i * _CH, _CH)], dstv)
            pltpu.sync_copy(onesv, tbl.at[dstv], add=True)
        plsc.subcore_barrier()
        pltpu.sync_copy(tbl.at[pl.ds(r0, _SLAB)], deg_hbm.at[pl.ds(r0, _SLAB)])


_deg_kernel = pl.kernel(
    _deg_body,
    out_type=jax.ShapeDtypeStruct((_NP, 1), jnp.float32),
    mesh=_mesh,
    scratch_types=[
        pltpu.VMEM_SHARED((_NP, 1), jnp.float32),
        pltpu.VMEM((_CH,), jnp.int32),
        pltpu.VMEM((_CH, 1), jnp.float32),
    ],
)


def _prop_body(h_hbm, src_hbm, dst_hbm, out_hbm, acc, srcv, dstv, rows, sem):
    c = lax.axis_index("c")
    s = lax.axis_index("s")
    r0 = s * _SLAB
    base = c * _NP + r0
    pltpu.sync_copy(h_hbm.at[pl.ds(base, _SLAB)], acc.at[pl.ds(r0, _SLAB)])
    plsc.subcore_barrier()
    e0 = c * _E + s * _EPT
    ed0 = s * _EPT
    for i in range(_NCH):
        pltpu.sync_copy(src_hbm.at[pl.ds(e0 + i * _CH, _CH)], srcv)
        pltpu.sync_copy(dst_hbm.at[pl.ds(ed0 + i * _CH, _CH)], dstv)
        pltpu.async_copy(h_hbm.at[srcv], rows, sem).wait()
        pltpu.sync_copy(rows, acc.at[dstv], add=True)
    plsc.subcore_barrier()
    pltpu.sync_copy(acc.at[pl.ds(r0, _SLAB)], out_hbm.at[pl.ds(base, _SLAB)])


_prop_kernel = pl.kernel(
    _prop_body,
    out_type=jax.ShapeDtypeStruct((2 * _NP, _H), jnp.float32),
    mesh=_mesh,
    scratch_types=[
        pltpu.VMEM_SHARED((_NP, _H), jnp.float32),
        pltpu.VMEM((_CH,), jnp.int32),
        pltpu.VMEM((_CH,), jnp.int32),
        pltpu.VMEM((_CH, _H), jnp.float32),
        pltpu.SemaphoreType.DMA,
    ],
)


def _tca_body(x_ref, deg_ref, w_ref, h_ref, dinv_ref):
    dinv = lax.rsqrt(deg_ref[...] + 1.0)
    h = jnp.dot(x_ref[...], w_ref[...], preferred_element_type=jnp.float32) * dinv
    dinv_ref[...] = dinv
    h_ref[0] = h[:, :_H]
    h_ref[1] = h[:, _H:]


def _tcb_body(o_ref, dinv_ref, b_ref, w_ref, h_ref):
    dinv = dinv_ref[...]
    z = jnp.concatenate([o_ref[0], o_ref[1]], axis=1) * dinv + b_ref[...]
    z = jnp.maximum(z, 0.0)
    h = jnp.dot(z, w_ref[...], preferred_element_type=jnp.float32) * dinv
    h_ref[0] = h[:, :_H]
    h_ref[1] = h[:, _H:]


def _tcc_body(o_ref, dinv_ref, b_ref, batch_ref, wfc_ref, bfc_ref, y_ref,
              sums_ref, cnts_ref):
    i = pl.program_id(0)

    @pl.when(i == 0)
    def _():
        sums_ref[...] = jnp.zeros_like(sums_ref)
        cnts_ref[...] = jnp.zeros_like(cnts_ref)

    z = jnp.concatenate([o_ref[0], o_ref[1]], axis=1) * dinv_ref[...] + b_ref[...]
    z = jnp.maximum(z, 0.0)
    gid = lax.broadcasted_iota(jnp.int32, (_G, 1), 0)
    oh = (gid == batch_ref[...]).astype(jnp.float32)
    sums_ref[...] += jnp.dot(oh, z, preferred_element_type=jnp.float32)
    cnts_ref[...] += jnp.sum(oh, axis=1, keepdims=True)

    @pl.when(i == pl.num_programs(0) - 1)
    def _():
        y = jnp.dot(sums_ref[...], wfc_ref[...], preferred_element_type=jnp.float32)
        y_ref[...] = y / jnp.maximum(cnts_ref[...], 1.0) + bfc_ref[...]


_tca = pl.pallas_call(
    _tca_body,
    grid=(_NBLK,),
    in_specs=[
        pl.BlockSpec((_BN, _D), lambda i: (i, 0)),
        pl.BlockSpec((_BN, 1), lambda i: (i, 0)),
        pl.BlockSpec((_D, _D), lambda i: (0, 0)),
    ],
    out_specs=[
        pl.BlockSpec((2, _BN, _H), lambda i: (0, i, 0)),
        pl.BlockSpec((_BN, 1), lambda i: (i, 0)),
    ],
    out_shape=[
        jax.ShapeDtypeStruct((2, _NP, _H), jnp.float32),
        jax.ShapeDtypeStruct((_NP, 1), jnp.float32),
    ],
)

_tcb = pl.pallas_call(
    _tcb_body,
    grid=(_NBLK,),
    in_specs=[
        pl.BlockSpec((2, _BN, _H), lambda i: (0, i, 0)),
        pl.BlockSpec((_BN, 1), lambda i: (i, 0)),
        pl.BlockSpec((1, _D), lambda i: (0, 0)),
        pl.BlockSpec((_D, _D), lambda i: (0, 0)),
    ],
    out_specs=pl.BlockSpec((2, _BN, _H), lambda i: (0, i, 0)),
    out_shape=jax.ShapeDtypeStruct((2, _NP, _H), jnp.float32),
)

_tcc = pl.pallas_call(
    _tcc_body,
    grid=(_NBLK,),
    in_specs=[
        pl.BlockSpec((2, _BN, _H), lambda i: (0, i, 0)),
        pl.BlockSpec((_BN, 1), lambda i: (i, 0)),
        pl.BlockSpec((1, _D), lambda i: (0, 0)),
        pl.BlockSpec((1, _BN), lambda i: (0, i)),
        pl.BlockSpec((_D, 1), lambda i: (0, 0)),
        pl.BlockSpec((1, 1), lambda i: (0, 0)),
    ],
    out_specs=pl.BlockSpec((_G, 1), lambda i: (0, 0)),
    out_shape=jax.ShapeDtypeStruct((_G, 1), jnp.float32),
    scratch_shapes=[
        pltpu.VMEM((_G, _D), jnp.float32),
        pltpu.VMEM((_G, 1), jnp.float32),
    ],
)


def kernel(x, edge_index, batch, W1, b1, W2, b2, Wfc, bfc):
    src = edge_index[0]
    dst = edge_index[1]
    srcs = jnp.concatenate([src, src + _NP])
    x_pad = jnp.zeros((_NP, _D), jnp.float32).at[:_N].set(x)
    batch_pad = jnp.full((1, _NP), -1, jnp.int32).at[0, :_N].set(batch)
    ones_ch = jnp.ones((_CH, 1), jnp.float32)
    zeros_np = jnp.zeros((_NP, 1), jnp.float32)

    deg = _deg_kernel(dst, ones_ch, zeros_np)
    h1, dinv = _tca(x_pad, deg, W1)
    out1 = _prop_kernel(h1.reshape(2 * _NP, _H), srcs, dst)
    h2 = _tcb(out1.reshape(2, _NP, _H), dinv, b1.reshape(1, _D), W2)
    out2 = _prop_kernel(h2.reshape(2 * _NP, _H), srcs, dst)
    y = _tcc(out2.reshape(2, _NP, _H), dinv, b2.reshape(1, _D),
             batch_pad, Wfc, bfc.reshape(1, 1))
    return y


# speedup vs baseline: 21.9204x; 21.9204x over previous
import functools

import jax
import jax.numpy as jnp
from jax import lax
from jax.experimental import pallas as pl
from jax.experimental.pallas import tpu as pltpu
from jax.experimental.pallas import tpu_sc as plsc

_N = 10000
_E = 320000
_D = 128
_G = 64
_H = 64
_NP = 10240
_BN = 1024
_NBLK = _NP // _BN
_NSUB = 16
_SLAB = _NP // _NSUB
_EPT = _E // _NSUB
_CH = 1000
_NCH = _EPT // _CH

_mesh = plsc.VectorSubcoreMesh(core_axis_name="c", subcore_axis_name="s")


def _deg_body(dst_hbm, ones_hbm, zeros_hbm, deg_hbm, tbl, dstv, onesv):
    c = lax.axis_index("c")
    s = lax.axis_index("s")

    @pl.when(c == 0)
    def _():
        r0 = s * _SLAB
        pltpu.sync_copy(zeros_hbm.at[pl.ds(r0, _SLAB)], tbl.at[pl.ds(r0, _SLAB)])
        pltpu.sync_copy(ones_hbm, onesv)
        plsc.subcore_barrier()
        e0 = s * _EPT
        for i in range(_NCH):
            pltpu.sync_copy(dst_hbm.at[pl.ds(e0 + i * _CH, _CH)], dstv)
            pltpu.sync_copy(onesv, tbl.at[dstv], add=True)
        plsc.subcore_barrier()
        pltpu.sync_copy(tbl.at[pl.ds(r0, _SLAB)], deg_hbm.at[pl.ds(r0, _SLAB)])


_DW = 16

_deg_kernel = pl.kernel(
    _deg_body,
    out_type=jax.ShapeDtypeStruct((_NP, _DW), jnp.float32),
    mesh=_mesh,
    scratch_types=[
        pltpu.VMEM_SHARED((_NP, _DW), jnp.float32),
        pltpu.VMEM((_CH,), jnp.int32),
        pltpu.VMEM((_CH, _DW), jnp.float32),
    ],
    compiler_params=pltpu.CompilerParams(use_tc_tiling_on_sc=False),
)


def _prop_body(h_hbm, src_hbm, dst_hbm, out_hbm, acc, srcv, dstv, rows, sem):
    c = lax.axis_index("c")
    s = lax.axis_index("s")
    r0 = s * _SLAB
    base = c * _NP + r0
    pltpu.sync_copy(h_hbm.at[pl.ds(base, _SLAB)], acc.at[pl.ds(r0, _SLAB)])
    plsc.subcore_barrier()
    e0 = c * _E + s * _EPT
    ed0 = s * _EPT
    for i in range(_NCH):
        pltpu.sync_copy(src_hbm.at[pl.ds(e0 + i * _CH, _CH)], srcv)
        pltpu.sync_copy(dst_hbm.at[pl.ds(ed0 + i * _CH, _CH)], dstv)
        pltpu.async_copy(h_hbm.at[srcv], rows, sem).wait()
        pltpu.sync_copy(rows, acc.at[dstv], add=True)
    plsc.subcore_barrier()
    pltpu.sync_copy(acc.at[pl.ds(r0, _SLAB)], out_hbm.at[pl.ds(base, _SLAB)])


_prop_kernel = pl.kernel(
    _prop_body,
    out_type=jax.ShapeDtypeStruct((2 * _NP, _H), jnp.float32),
    mesh=_mesh,
    scratch_types=[
        pltpu.VMEM_SHARED((_NP, _H), jnp.float32),
        pltpu.VMEM((_CH,), jnp.int32),
        pltpu.VMEM((_CH,), jnp.int32),
        pltpu.VMEM((_CH, _H), jnp.float32),
        pltpu.SemaphoreType.DMA,
    ],
    compiler_params=pltpu.CompilerParams(use_tc_tiling_on_sc=False),
)


def _tca_body(x_ref, deg_ref, w_ref, h_ref, dinv_ref):
    dinv = lax.rsqrt(deg_ref[:, :1] + 1.0)
    h = jnp.dot(x_ref[...], w_ref[...], preferred_element_type=jnp.float32, precision=lax.Precision.HIGHEST) * dinv
    dinv_ref[...] = dinv
    h_ref[0] = h[:, :_H]
    h_ref[1] = h[:, _H:]


def _tcb_body(o_ref, dinv_ref, b_ref, w_ref, h_ref):
    dinv = dinv_ref[...]
    z = jnp.concatenate([o_ref[0], o_ref[1]], axis=1) * dinv + b_ref[...]
    z = jnp.maximum(z, 0.0)
    h = jnp.dot(z.astype(jnp.bfloat16), w_ref[...].astype(jnp.bfloat16),
                preferred_element_type=jnp.float32) * dinv
    h_ref[0] = h[:, :_H]
    h_ref[1] = h[:, _H:]


def _tcc_body(o_ref, dinv_ref, b_ref, batch_ref, wfc_ref, bfc_ref, y_ref,
              sums_ref, cnts_ref):
    i = pl.program_id(0)

    @pl.when(i == 0)
    def _():
        sums_ref[...] = jnp.zeros_like(sums_ref)
        cnts_ref[...] = jnp.zeros_like(cnts_ref)

    z = jnp.concatenate([o_ref[0], o_ref[1]], axis=1) * dinv_ref[...] + b_ref[...]
    z = jnp.maximum(z, 0.0)
    gid = lax.broadcasted_iota(jnp.int32, (_G, 1), 0)
    oh = (gid == batch_ref[...]).astype(jnp.float32)
    sums_ref[...] += jnp.dot(oh, z, preferred_element_type=jnp.float32, precision=lax.Precision.HIGHEST)
    cnts_ref[...] += jnp.sum(oh, axis=1, keepdims=True)

    @pl.when(i == pl.num_programs(0) - 1)
    def _():
        pooled = sums_ref[...] / jnp.maximum(cnts_ref[...], 1.0)
        y = jnp.dot(pooled.astype(jnp.bfloat16), wfc_ref[...].astype(jnp.bfloat16),
                    preferred_element_type=jnp.float32)
        y_ref[...] = y + bfc_ref[...]


_tca = pl.pallas_call(
    _tca_body,
    grid=(_NBLK,),
    in_specs=[
        pl.BlockSpec((_BN, _D), lambda i: (i, 0)),
        pl.BlockSpec((_BN, _DW), lambda i: (i, 0)),
        pl.BlockSpec((_D, _D), lambda i: (0, 0)),
    ],
    out_specs=[
        pl.BlockSpec((2, _BN, _H), lambda i: (0, i, 0)),
        pl.BlockSpec((_BN, 1), lambda i: (i, 0)),
    ],
    out_shape=[
        jax.ShapeDtypeStruct((2, _NP, _H), jnp.float32),
        jax.ShapeDtypeStruct((_NP, 1), jnp.float32),
    ],
)

_tcb = pl.pallas_call(
    _tcb_body,
    grid=(_NBLK,),
    in_specs=[
        pl.BlockSpec((2, _BN, _H), lambda i: (0, i, 0)),
        pl.BlockSpec((_BN, 1), lambda i: (i, 0)),
        pl.BlockSpec((1, _D), lambda i: (0, 0)),
        pl.BlockSpec((_D, _D), lambda i: (0, 0)),
    ],
    out_specs=pl.BlockSpec((2, _BN, _H), lambda i: (0, i, 0)),
    out_shape=jax.ShapeDtypeStruct((2, _NP, _H), jnp.float32),
)

_tcc = pl.pallas_call(
    _tcc_body,
    grid=(_NBLK,),
    in_specs=[
        pl.BlockSpec((2, _BN, _H), lambda i: (0, i, 0)),
        pl.BlockSpec((_BN, 1), lambda i: (i, 0)),
        pl.BlockSpec((1, _D), lambda i: (0, 0)),
        pl.BlockSpec((1, _BN), lambda i: (0, i)),
        pl.BlockSpec((_D, 1), lambda i: (0, 0)),
        pl.BlockSpec((1, 1), lambda i: (0, 0)),
    ],
    out_specs=pl.BlockSpec((_G, 1), lambda i: (0, 0)),
    out_shape=jax.ShapeDtypeStruct((_G, 1), jnp.float32),
    scratch_shapes=[
        pltpu.VMEM((_G, _D), jnp.float32),
        pltpu.VMEM((_G, 1), jnp.float32),
    ],
)


def kernel(x, edge_index, batch, W1, b1, W2, b2, Wfc, bfc):
    src = edge_index[0]
    dst = edge_index[1]
    srcs = jnp.concatenate([src, src + _NP])
    x_pad = jnp.zeros((_NP, _D), jnp.float32).at[:_N].set(x)
    batch_pad = jnp.full((1, _NP), -1, jnp.int32).at[0, :_N].set(batch)
    ones_ch = jnp.ones((_CH, _DW), jnp.float32)
    zeros_np = jnp.zeros((_NP, _DW), jnp.float32)

    deg = _deg_kernel(dst, ones_ch, zeros_np)
    h1, dinv = _tca(x_pad, deg, W1)
    out1 = _prop_kernel(h1.reshape(2 * _NP, _H), srcs, dst)
    h2 = _tcb(out1.reshape(2, _NP, _H), dinv, b1.reshape(1, _D), W2)
    out2 = _prop_kernel(h2.reshape(2 * _NP, _H), srcs, dst)
    y = _tcc(out2.reshape(2, _NP, _H), dinv, b2.reshape(1, _D),
             batch_pad, Wfc, bfc.reshape(1, 1))
    return y


# speedup vs baseline: 21.9360x; 1.0007x over previous
import jax
import jax.numpy as jnp
from jax import lax
from jax.experimental import pallas as pl
from jax.experimental.pallas import tpu as pltpu
from jax.experimental.pallas import tpu_sc as plsc

_N = 10000
_E = 320000
_D = 128
_G = 64
_H = 64
_NP = 10240
_BN = 1024
_NBLK = _NP // _BN
_NSUB = 16
_SLAB = _NP // _NSUB
_EPT = _E // _NSUB
_CH = 1000
_NCH = _EPT // _CH

_mesh = plsc.VectorSubcoreMesh(core_axis_name="c", subcore_axis_name="s")


def _deg_body(dst_hbm, ones_hbm, zeros_hbm, deg_hbm, tbl, dstv, onesv):
    c = lax.axis_index("c")
    s = lax.axis_index("s")

    @pl.when(c == 0)
    def _():
        r0 = s * _SLAB
        pltpu.sync_copy(zeros_hbm.at[pl.ds(r0, _SLAB)], tbl.at[pl.ds(r0, _SLAB)])
        pltpu.sync_copy(ones_hbm, onesv)
        plsc.subcore_barrier()
        e0 = s * _EPT
        for i in range(_NCH):
            pltpu.sync_copy(dst_hbm.at[pl.ds(e0 + i * _CH, _CH)], dstv)
            pltpu.sync_copy(onesv, tbl.at[dstv], add=True)
        plsc.subcore_barrier()
        pltpu.sync_copy(tbl.at[pl.ds(r0, _SLAB)], deg_hbm.at[pl.ds(r0, _SLAB)])


_DW = 16

_deg_kernel = pl.kernel(
    _deg_body,
    out_type=jax.ShapeDtypeStruct((_NP, _DW), jnp.float32),
    mesh=_mesh,
    scratch_types=[
        pltpu.VMEM_SHARED((_NP, _DW), jnp.float32),
        pltpu.VMEM((_CH,), jnp.int32),
        pltpu.VMEM((_CH, _DW), jnp.float32),
    ],
    compiler_params=pltpu.CompilerParams(use_tc_tiling_on_sc=False),
)


def _prop_body(h_hbm, src_hbm, dst_hbm, out_hbm, acc, srcv, dstv, rows, sem):
    c = lax.axis_index("c")
    s = lax.axis_index("s")
    r0 = s * _SLAB
    base = c * _NP + r0
    pltpu.sync_copy(h_hbm.at[pl.ds(base, _SLAB)], acc.at[pl.ds(r0, _SLAB)])
    plsc.subcore_barrier()
    e0 = c * _E + s * _EPT
    ed0 = s * _EPT
    for i in range(_NCH):
        pltpu.sync_copy(src_hbm.at[pl.ds(e0 + i * _CH, _CH)], srcv)
        pltpu.sync_copy(dst_hbm.at[pl.ds(ed0 + i * _CH, _CH)], dstv)
        pltpu.async_copy(h_hbm.at[srcv], rows, sem).wait()
        pltpu.sync_copy(rows, acc.at[dstv], add=True)
    plsc.subcore_barrier()
    pltpu.sync_copy(acc.at[pl.ds(r0, _SLAB)], out_hbm.at[pl.ds(base, _SLAB)])


_prop_kernel = pl.kernel(
    _prop_body,
    out_type=jax.ShapeDtypeStruct((2 * _NP, _H), jnp.float32),
    mesh=_mesh,
    scratch_types=[
        pltpu.VMEM_SHARED((_NP, _H), jnp.float32),
        pltpu.VMEM((_CH,), jnp.int32),
        pltpu.VMEM((_CH,), jnp.int32),
        pltpu.VMEM((_CH, _H), jnp.float32),
        pltpu.SemaphoreType.DMA,
    ],
    compiler_params=pltpu.CompilerParams(use_tc_tiling_on_sc=False),
)


def _tca_body(x_ref, deg_ref, w_ref, h_ref, dinv_ref):
    dinv = lax.rsqrt(deg_ref[:, :1] + 1.0)
    h = jnp.dot(x_ref[...], w_ref[...], preferred_element_type=jnp.float32, precision=lax.Precision.HIGHEST) * dinv
    dinv_ref[...] = dinv
    h_ref[0] = h[:, :_H]
    h_ref[1] = h[:, _H:]


def _tcb_body(o_ref, dinv_ref, b_ref, w_ref, h_ref):
    dinv = dinv_ref[...]
    z = jnp.concatenate([o_ref[0], o_ref[1]], axis=1) * dinv + b_ref[...]
    z = jnp.maximum(z, 0.0)
    h = jnp.dot(z.astype(jnp.bfloat16), w_ref[...].astype(jnp.bfloat16),
                preferred_element_type=jnp.float32) * dinv
    h_ref[0] = h[:, :_H]
    h_ref[1] = h[:, _H:]


def _tcc_body(o_ref, dinv_ref, b_ref, batch_ref, wfc_ref, bfc_ref, y_ref,
              sums_ref, cnts_ref):
    i = pl.program_id(0)

    @pl.when(i == 0)
    def _():
        sums_ref[...] = jnp.zeros_like(sums_ref)
        cnts_ref[...] = jnp.zeros_like(cnts_ref)

    z = jnp.concatenate([o_ref[0], o_ref[1]], axis=1) * dinv_ref[...] + b_ref[...]
    z = jnp.maximum(z, 0.0)
    gid = lax.broadcasted_iota(jnp.int32, (_G, 1), 0)
    oh = (gid == batch_ref[...]).astype(jnp.float32)
    sums_ref[...] += jnp.dot(oh, z, preferred_element_type=jnp.float32, precision=lax.Precision.HIGHEST)
    cnts_ref[...] += jnp.sum(oh, axis=1, keepdims=True)

    @pl.when(i == pl.num_programs(0) - 1)
    def _():
        pooled = sums_ref[...] / jnp.maximum(cnts_ref[...], 1.0)
        y = jnp.dot(pooled.astype(jnp.bfloat16), wfc_ref[...].astype(jnp.bfloat16),
                    preferred_element_type=jnp.float32)
        y_ref[...] = y + bfc_ref[...]


_tca = pl.pallas_call(
    _tca_body,
    grid=(_NBLK,),
    in_specs=[
        pl.BlockSpec((_BN, _D), lambda i: (i, 0)),
        pl.BlockSpec((_BN, _DW), lambda i: (i, 0)),
        pl.BlockSpec((_D, _D), lambda i: (0, 0)),
    ],
    out_specs=[
        pl.BlockSpec((2, _BN, _H), lambda i: (0, i, 0)),
        pl.BlockSpec((_BN, 1), lambda i: (i, 0)),
    ],
    out_shape=[
        jax.ShapeDtypeStruct((2, _NP, _H), jnp.float32),
        jax.ShapeDtypeStruct((_NP, 1), jnp.float32),
    ],
)

_tcb = pl.pallas_call(
    _tcb_body,
    grid=(_NBLK,),
    in_specs=[
        pl.BlockSpec((2, _BN, _H), lambda i: (0, i, 0)),
        pl.BlockSpec((_BN, 1), lambda i: (i, 0)),
        pl.BlockSpec((1, _D), lambda i: (0, 0)),
        pl.BlockSpec((_D, _D), lambda i: (0, 0)),
    ],
    out_specs=pl.BlockSpec((2, _BN, _H), lambda i: (0, i, 0)),
    out_shape=jax.ShapeDtypeStruct((2, _NP, _H), jnp.float32),
)

_tcc = pl.pallas_call(
    _tcc_body,
    grid=(_NBLK,),
    in_specs=[
        pl.BlockSpec((2, _BN, _H), lambda i: (0, i, 0)),
        pl.BlockSpec((_BN, 1), lambda i: (i, 0)),
        pl.BlockSpec((1, _D), lambda i: (0, 0)),
        pl.BlockSpec((1, _BN), lambda i: (0, i)),
        pl.BlockSpec((_D, 1), lambda i: (0, 0)),
        pl.BlockSpec((1, 1), lambda i: (0, 0)),
    ],
    out_specs=pl.BlockSpec((_G, 1), lambda i: (0, 0)),
    out_shape=jax.ShapeDtypeStruct((_G, 1), jnp.float32),
    scratch_shapes=[
        pltpu.VMEM((_G, _D), jnp.float32),
        pltpu.VMEM((_G, 1), jnp.float32),
    ],
)


def kernel(x, edge_index, batch, W1, b1, W2, b2, Wfc, bfc):
    src = edge_index[0]
    dst = edge_index[1]
    srcs = jnp.concatenate([src, src + _NP])
    x_pad = jnp.zeros((_NP, _D), jnp.float32).at[:_N].set(x)
    batch_pad = jnp.full((1, _NP), -1, jnp.int32).at[0, :_N].set(batch)
    ones_ch = jnp.ones((_CH, _DW), jnp.float32)
    zeros_np = jnp.zeros((_NP, _DW), jnp.float32)

    deg = _deg_kernel(dst, ones_ch, zeros_np)
    h1, dinv = _tca(x_pad, deg, W1)
    out1 = _prop_kernel(h1.reshape(2 * _NP, _H), srcs, dst)
    h2 = _tcb(out1.reshape(2, _NP, _H), dinv, b1.reshape(1, _D), W2)
    out2 = _prop_kernel(h2.reshape(2 * _NP, _H), srcs, dst)
    y = _tcc(out2.reshape(2, _NP, _H), dinv, b2.reshape(1, _D),
             batch_pad, Wfc, bfc.reshape(1, 1))
    return y
